# baseline (speedup 1.0000x reference)
import functools

import jax
import jax.numpy as jnp
from jax import lax
from jax.experimental import pallas as pl
from jax.experimental.pallas import tpu as pltpu
from jax.experimental.pallas import tpu_sc as plsc

F = 2
B = 16384
H = 20
D = 32
V = 1000000
N = B * H
NC = 2
NS = 16
NW = NC * NS
PER_W = N // NW
CH = 1024
NCH = PER_W // CH
NBUF = 3

_mesh = plsc.VectorSubcoreMesh(core_axis_name="c", subcore_axis_name="s")


@functools.partial(
    pl.kernel,
    mesh=_mesh,
    compiler_params=pltpu.CompilerParams(use_tc_tiling_on_sc=False),
    out_type=jax.ShapeDtypeStruct((N, D), jnp.float32),
    scratch_types=[
        pltpu.VMEM((NBUF, CH), jnp.int32),
        pltpu.VMEM((NBUF, CH, D), jnp.float32),
        pltpu.SemaphoreType.DMA,
        pltpu.SemaphoreType.DMA,
        pltpu.SemaphoreType.DMA,
    ],
)
def _lookup1(idx_hbm, tab_hbm, out_hbm, idx_v, rows_v, sem0, sem1, sem2):
    wid = lax.axis_index("s") * NC + lax.axis_index("c")
    base = wid * PER_W
    sems = (sem0, sem1, sem2)

    def start(j):
        slot = j % NBUF
        off = base + j * CH
        pltpu.sync_copy(idx_hbm.at[pl.ds(off, CH)], idx_v.at[slot])
        return pltpu.async_copy(
            tab_hbm.at[idx_v.at[slot]], rows_v.at[slot], sems[slot])

    inflight = {j: start(j) for j in range(NBUF - 1)}
    for j in range(NCH):
        if j + NBUF - 1 < NCH:
            inflight[j + NBUF - 1] = start(j + NBUF - 1)
        inflight[j].wait()
        slot = j % NBUF
        pltpu.sync_copy(rows_v.at[slot], out_hbm.at[pl.ds(base + j * CH, CH)])


RPS = H * D // 128
FBB = 8
FW = 16
FSAMP = B // FW
FCH = FSAMP // FBB
FPAIRS = FCH // 2


@functools.partial(
    pl.kernel,
    mesh=_mesh,
    out_type=jax.ShapeDtypeStruct((F, B, H, D), jnp.float32),
    scratch_types=[
        pltpu.VMEM((2, FBB * RPS, 128), jnp.float32),
        pltpu.VMEM((2, FBB, H, D), jnp.float32),
        pltpu.SemaphoreType.DMA,
        pltpu.SemaphoreType.DMA,
        pltpu.SemaphoreType.DMA,
        pltpu.SemaphoreType.DMA,
    ],
)
def _format(y0_hbm, y1_hbm, out_hbm, in_v, out_v,
            isem_a, isem_b, osem_a, osem_b):
    wid = lax.axis_index("s") * NC + lax.axis_index("c")
    isems = (isem_a, isem_b)
    osems = (osem_a, osem_b)

    def run(y, f, w):
        def b0(c):
            return w * FSAMP + c * FBB

        def fire_in(slot, c):
            pltpu.async_copy(y.at[pl.ds(b0(c) * RPS, FBB * RPS)],
                             in_v.at[slot], isems[slot])

        def proc(slot, c):
            pltpu.make_async_copy(y.at[pl.ds(0, FBB * RPS)],
                                  in_v.at[slot], isems[slot]).wait()

            @pl.when(c >= 2)
            def _():
                pltpu.make_async_copy(out_hbm.at[f, pl.ds(0, FBB)],
                                      out_v.at[slot], osems[slot]).wait()

            def sample(s, carry):
                for h in range(H):
                    for c2 in range(2):
                        out_v[slot, s, h, pl.ds(16 * c2, 16)] = (
                            in_v[slot, RPS * s + h // 4,
                                 pl.ds(32 * (h % 4) + 16 * c2, 16)])
                return carry

            lax.fori_loop(0, FBB, sample, 0)
            pltpu.async_copy(out_v.at[slot],
                             out_hbm.at[f, pl.ds(b0(c), FBB)], osems[slot])

        fire_in(0, 0)

        def pair(jp, carry):
            c1 = 2 * jp + 1
            fire_in(1, c1)
            proc(0, c1 - 1)

            @pl.when(c1 + 1 < FCH)
            def _():
                fire_in(0, c1 + 1)

            proc(1, c1)
            return carry

        lax.fori_loop(0, FPAIRS, pair, 0)

        for slot in range(2):
            pltpu.make_async_copy(out_hbm.at[f, pl.ds(0, FBB)],
                                  out_v.at[slot], osems[slot]).wait()

    @pl.when(wid < FW)
    def _():
        run(y0_hbm, 0, wid)

    @pl.when(wid >= FW)
    def _():
        run(y1_hbm, 1, wid - FW)


def kernel(indices, table_0, table_1):
    i0 = indices[0].reshape(N).astype(jnp.int32)
    i1 = indices[1].reshape(N).astype(jnp.int32)
    o0 = _lookup1(i0, table_0)
    o1 = _lookup1(i1, table_1)
    return _format(o0.reshape(N * D // 128, 128), o1.reshape(N * D // 128, 128))


# speedup vs baseline: 1.0351x; 1.0351x over previous
import functools

import jax
import jax.numpy as jnp
from jax import lax
from jax.experimental import pallas as pl
from jax.experimental.pallas import tpu as pltpu
from jax.experimental.pallas import tpu_sc as plsc

F = 2
B = 16384
H = 20
D = 32
V = 1000000
N = B * H
NC = 2
NS = 16
NW = NC * NS
PER_W = N // NW
CH = 1024
NCH = PER_W // CH
NBUF = 3

_mesh = plsc.VectorSubcoreMesh(core_axis_name="c", subcore_axis_name="s")


@functools.partial(
    pl.kernel,
    mesh=_mesh,
    compiler_params=pltpu.CompilerParams(use_tc_tiling_on_sc=False),
    out_type=jax.ShapeDtypeStruct((N, D), jnp.float32),
    scratch_types=[
        pltpu.VMEM((NBUF, CH), jnp.int32),
        pltpu.VMEM((NBUF, CH, D), jnp.float32),
        pltpu.SemaphoreType.DMA,
        pltpu.SemaphoreType.DMA,
        pltpu.SemaphoreType.DMA,
    ],
)
def _lookup1(idx_hbm, tab_hbm, out_hbm, idx_v, rows_v, sem0, sem1, sem2):
    wid = lax.axis_index("s") * NC + lax.axis_index("c")
    base = wid * PER_W
    sems = (sem0, sem1, sem2)

    def start(j):
        slot = j % NBUF
        off = base + j * CH
        pltpu.sync_copy(idx_hbm.at[pl.ds(off, CH)], idx_v.at[slot])
        return pltpu.async_copy(
            tab_hbm.at[idx_v.at[slot]], rows_v.at[slot], sems[slot])

    inflight = {j: start(j) for j in range(NBUF - 1)}
    for j in range(NCH):
        if j + NBUF - 1 < NCH:
            inflight[j + NBUF - 1] = start(j + NBUF - 1)
        inflight[j].wait()
        slot = j % NBUF
        pltpu.sync_copy(rows_v.at[slot], out_hbm.at[pl.ds(base + j * CH, CH)])


def kernel(indices, table_0, table_1):
    i0 = indices[0].reshape(N).astype(jnp.int32)
    i1 = indices[1].reshape(N).astype(jnp.int32)
    o0 = _lookup1(i0, table_0)
    o1 = _lookup1(i1, table_1)
    return jnp.stack([o0.reshape(B, H, D), o1.reshape(B, H, D)], axis=0)
